# baseline (speedup 1.0000x reference)
import functools

import jax
import jax.numpy as jnp
from jax import lax
from jax.experimental import pallas as pl
from jax.experimental.pallas import tpu as pltpu
from jax.experimental.pallas import tpu_sc as plsc

_NC = 2
_NS = 16
_NW = _NC * _NS
_NBUF = 8
_PD = 128
_TC = 320
_L = 16


def _make_transpose(v, dim):
    n_chunks = v // _TC
    per_w = -(-n_chunks // _NW)
    n_groups = per_w // 2
    mesh = plsc.VectorSubcoreMesh(core_axis_name="c", subcore_axis_name="s")

    @functools.partial(
        pl.kernel,
        mesh=mesh,
        out_type=jax.ShapeDtypeStruct((v, _PD), jnp.float32),
        scratch_types=[
            pltpu.VMEM((dim, _TC), jnp.float32),
            pltpu.VMEM((dim, _TC), jnp.float32),
            pltpu.VMEM((_TC, dim), jnp.float32),
            pltpu.VMEM((_TC, dim), jnp.float32),
            pltpu.SemaphoreType.DMA((2,)),
            pltpu.SemaphoreType.DMA((2,)),
        ],
        compiler_params=pltpu.CompilerParams(
            use_tc_tiling_on_sc=False, needs_layout_passes=False),
    )
    def k(tt_hbm, out_hbm, in0, in1, ou0, ou1, isem, wsem):
        wid = lax.axis_index("s") * _NC + lax.axis_index("c")
        ins, ous = [in0, in1], [ou0, ou1]
        lanes = lax.iota(jnp.int32, _L)

        def chunk_col(j):
            return lax.min(wid + _NW * j, n_chunks - 1) * _TC

        def in_copy(j, s):
            return pltpu.make_async_copy(
                tt_hbm.at[:, pl.ds(chunk_col(j), _TC)], ins[s], isem.at[s])

        def out_copy(j, s):
            return pltpu.make_async_copy(
                ous[s], out_hbm.at[pl.ds(chunk_col(j), _TC), pl.ds(0, dim)],
                wsem.at[s])

        def transpose_block(s):
            ib, ob = ins[s], ous[s]

            def body(i, carry):
                col = jnp.full((_L,), i, jnp.int32)
                for d0 in range(0, dim, _L):
                    vals = plsc.load_gather(ib, [lanes + d0, col])
                    ob[i, pl.ds(d0, _L)] = vals
                return carry

            lax.fori_loop(0, _TC, body, 0)

        in_copy(0, 0).start()
        in_copy(1, 1).start()

        def group(g, carry):
            for s in range(2):
                c = 2 * g + s
                in_copy(c, s).wait()
                transpose_block(s)
                out_copy(c, s).start()
                out_copy(c, s).wait()
                in_copy(c + 2, s).start()
            return carry

        lax.fori_loop(0, n_groups - 1, group, 0)

        for s in range(2):
            c = 2 * (n_groups - 1) + s
            in_copy(c, s).wait()
            transpose_block(s)
            out_copy(c, s).start()
            out_copy(c, s).wait()

    return k


def _make_gather(b, l, dim):
    rows_per_w = b // _NW
    n_groups = rows_per_w // _NBUF
    mesh = plsc.VectorSubcoreMesh(core_axis_name="c", subcore_axis_name="s")

    @functools.partial(
        pl.kernel,
        mesh=mesh,
        out_type=jax.ShapeDtypeStruct((b, l, _PD), jnp.float32),
        scratch_types=[
            pltpu.VMEM((rows_per_w, l), jnp.int32),
            pltpu.VMEM((_NBUF, l, dim), jnp.float32),
            pltpu.SemaphoreType.DMA((_NBUF,)),
            pltpu.SemaphoreType.DMA((_NBUF,)),
        ],
        compiler_params=pltpu.CompilerParams(use_tc_tiling_on_sc=False),
    )
    def k(idx_hbm, table_hbm, out_hbm, idx_v, bufs, gsem, wsem):
        wid = lax.axis_index("s") * _NC + lax.axis_index("c")
        r0 = wid * rows_per_w
        pltpu.sync_copy(idx_hbm.at[pl.ds(r0, rows_per_w)], idx_v)

        def gather_copy(r, s):
            return pltpu.make_async_copy(
                table_hbm.at[idx_v.at[r]], bufs.at[s], gsem.at[s])

        def write_copy(r, s):
            return pltpu.make_async_copy(
                bufs.at[pl.ds(s, 1)],
                out_hbm.at[pl.ds(r0 + r, 1), :, pl.ds(0, dim)],
                wsem.at[s])

        for s in range(_NBUF):
            gather_copy(s, s).start()

        def group(g, carry):
            row0 = g * _NBUF
            for s in range(_NBUF):
                r = row0 + s
                gather_copy(r, s).wait()
                write_copy(r, s).start()
                write_copy(r, s).wait()
                gather_copy(r + _NBUF, s).start()
            return carry

        lax.fori_loop(0, n_groups - 1, group, 0)

        row0 = (n_groups - 1) * _NBUF
        for s in range(_NBUF):
            r = row0 + s
            gather_copy(r, s).wait()
            write_copy(r, s).start()
            write_copy(r, s).wait()

    return k


def kernel(indices, table):
    b, l = indices.shape
    v, dim = table.shape
    padded = _make_transpose(v, dim)(table.T)
    flat = padded.reshape(v * (_PD // dim), dim)
    idx2 = indices.astype(jnp.int32) * (_PD // dim)
    res = _make_gather(b, l, dim)(idx2, flat)
    return res[:, :, :dim]
